# baseline (speedup 1.0000x reference)
import functools

import jax
import jax.numpy as jnp
import numpy as np
from jax import lax
from jax.experimental import pallas as pl
from jax.experimental.pallas import tpu as pltpu
from jax.experimental.pallas import tpu_sc as plsc

_NUM_CLASSES = 2
_NPTS = 100
_NSEL = 95
_B = 4
_KP = 80000
_G = 128
_HW = _G * _G
_CF = 512
_ROW = 528
_CHUNK = _KP // 8
_STEPS = _CHUNK // 16
_MSK31 = np.int32(0x7FFFFFFF)
_MIN32 = np.int32(-2147483648)
_MAX32 = np.int32(2147483647)


def _fixed_draws():
    k1, k2 = jax.random.split(jax.random.key(42))
    over = jax.random.uniform(k1, (_B, _KP, 2), dtype=jnp.float32)
    cov = jax.random.uniform(k2, (_B, _NPTS - _NSEL, 2), dtype=jnp.float32)
    return (np.asarray(over), np.asarray(cov))


try:
    with jax.default_device(jax.local_devices(backend="cpu")[0]):
        _OVER_NP, _COV_NP = _fixed_draws()
except Exception:
    try:
        _OVER_NP, _COV_NP = _fixed_draws()
    except Exception:
        _OVER_NP = np.zeros((_B, _KP, 2), np.float32)
        _COV_NP = np.zeros((_B, _NPTS - _NSEL, 2), np.float32)

_XS_NP = np.ascontiguousarray(_OVER_NP[..., 0]).reshape(-1)
_YS_NP = np.ascontiguousarray(_OVER_NP[..., 1]).reshape(-1)
_CVX_NP = np.zeros((_B, 8), np.float32)
_CVY_NP = np.zeros((_B, 8), np.float32)
_CVX_NP[:, : _NPTS - _NSEL] = _COV_NP[..., 0]
_CVY_NP[:, : _NPTS - _NSEL] = _COV_NP[..., 1]
_CVX_NP = _CVX_NP.reshape(-1)
_CVY_NP = _CVY_NP.reshape(-1)


def _splat_to_scalar(v):
    return jnp.max(v) if getattr(v, "ndim", 0) else v


def _sc_fused_body(ms_hbm, xs_hbm, ys_hbm, mk_hbm, ft_hbm, cx_hbm, cy_hbm,
                   px_hbm, py_hbm, rep_hbm,
                   ms_v, xs_v, ys_v, q_v, hist_v, hm_v,
                   m8_v, mi8_v, mx8_v, my8_v,
                   wq_v, wi_v, wx_v, wy_v,
                   cq_v, ci_v, cx_v, cy_v,
                   cnt16_v, c8_v, px_v, py_v, covx_v, covy_v,
                   pxw_v, pyw_v, ctab_v, idxb_v, gbuf_v, rep16_v, sem,
                   hist_sh, wq_sh, wi_sh, wx_sh, wy_sh, cnt_sh,
                   px_sh, py_sh):
    i32 = jnp.int32
    f32 = jnp.float32
    c = lax.axis_index("c")
    s = lax.axis_index("s")
    half = s // 8
    chunk = s % 8
    batch = c * 2 + half
    row0 = half * 8
    lanes = lax.iota(i32, 16)
    zeros16 = jnp.zeros((16,), i32)
    maxv16 = zeros16 + _MAX32

    ms_off = pl.multiple_of(batch * (2 * _HW), 8)
    pltpu.sync_copy(ms_hbm.at[pl.ds(ms_off, 2 * _HW)], ms_v)
    off_in = pl.multiple_of(batch * _KP + chunk * _CHUNK, 8)
    pltpu.sync_copy(xs_hbm.at[pl.ds(off_in, _CHUNK)], xs_v)
    pltpu.sync_copy(ys_hbm.at[pl.ds(off_in, _CHUNK)], ys_v)

    def tap_vectors(xo, yo):
        gx1 = (2.0 * xo - 1.0) + 1.0
        gy1 = (2.0 * yo - 1.0) + 1.0
        ix = (gx1 * 128.0 - 1.0) / 2.0
        iy = (gy1 * 128.0 - 1.0) / 2.0
        itx = ix.astype(i32)
        ity = iy.astype(i32)
        ix0 = itx - jnp.where(ix < itx.astype(f32), 1, 0)
        iy0 = ity - jnp.where(iy < ity.astype(f32), 1, 0)
        wx1 = ix - ix0.astype(f32)
        wx0 = 1.0 - wx1
        wy1 = iy - iy0.astype(f32)
        wy0 = 1.0 - wy1
        ix1 = ix0 + 1
        iy1 = iy0 + 1
        vx0 = (ix0 >= 0) & (ix0 <= _G - 1)
        vx1 = (ix1 >= 0) & (ix1 <= _G - 1)
        vy0 = (iy0 >= 0) & (iy0 <= _G - 1)
        vy1 = (iy1 >= 0) & (iy1 <= _G - 1)
        xc0 = jnp.clip(ix0, 0, _G - 1)
        xc1 = jnp.clip(ix1, 0, _G - 1)
        yc0 = jnp.clip(iy0, 0, _G - 1) * _G
        yc1 = jnp.clip(iy1, 0, _G - 1) * _G
        lin = (yc0 + xc0, yc0 + xc1, yc1 + xc0, yc1 + xc1)
        fv = (jnp.where(vx0 & vy0, 1.0, 0.0), jnp.where(vx1 & vy0, 1.0, 0.0),
              jnp.where(vx0 & vy1, 1.0, 0.0), jnp.where(vx1 & vy1, 1.0, 0.0))
        wv = (wx0 * wy0, wx1 * wy0, wx0 * wy1, wx1 * wy1)
        return lin, fv, wv

    def u_step(i, carry):
        o = i * 16
        xo = xs_v[pl.ds(o, 16)]
        yo = ys_v[pl.ds(o, 16)]
        lin, fv, wv = tap_vectors(xo, yo)

        def chan(off):
            a0 = (plsc.load_gather(ms_v, [lin[0] + off]) * fv[0]) * wv[0]
            a1 = (plsc.load_gather(ms_v, [lin[1] + off]) * fv[1]) * wv[1]
            a2 = (plsc.load_gather(ms_v, [lin[2] + off]) * fv[2]) * wv[2]
            a3 = (plsc.load_gather(ms_v, [lin[3] + off]) * fv[3]) * wv[3]
            return ((a0 + a1) + a2) + a3

        u = -1.0 * (chan(0) - chan(_HW))
        bb = lax.bitcast_convert_type(u, i32)
        aa = bb ^ (lax.shift_right_arithmetic(bb, 31) & _MSK31)
        qb = (~aa) ^ _MIN32
        q_v[pl.ds(o, 16)] = qb
        return carry

    lax.fori_loop(0, _STEPS, u_step, np.int32(0))

    t = np.int32(_NSEL)
    prefix = np.int32(0)
    for rnd in range(4):
        shift = 24 - 8 * rnd

        def zstep(j, carry):
            hist_v[pl.ds(j * 16, 16)] = zeros16
            return carry

        lax.fori_loop(0, 256, zstep, np.int32(0))

        def hstep(j, carry):
            qv = q_v[pl.ds(j * 16, 16)]
            binv = lax.shift_right_logical(qv, shift) & 255
            hidx = lanes * 256 + binv
            if rnd == 0:
                cnt = plsc.load_gather(hist_v, [hidx])
                plsc.store_scatter(hist_v, [hidx], cnt + 1)
            else:
                act = lax.shift_right_logical(qv, shift + 8) == carry
                cnt = plsc.load_gather(hist_v, [hidx], mask=act)
                plsc.store_scatter(hist_v, [hidx], cnt + 1, mask=act)
            return carry

        lax.fori_loop(0, _STEPS, hstep, prefix)

        def mstep(j, carry):
            acc = zeros16
            for l in range(16):
                acc = acc + hist_v[pl.ds(l * 256 + j * 16, 16)]
            hm_v[pl.ds(j * 16, 16)] = acc
            return carry

        lax.fori_loop(0, 16, mstep, np.int32(0))
        pltpu.sync_copy(hm_v, hist_sh.at[c, s])
        plsc.subcore_barrier()
        pltpu.sync_copy(hist_sh.at[c, pl.ds(row0, 8)], m8_v)

        def sstep(j, carry):
            total, found, bstar, cumbefore = carry
            g = zeros16
            for l in range(8):
                g = g + m8_v[l, pl.ds(j * 16, 16)]
            csum = plsc.cumsum(g)
            full = total + csum
            hitv = full >= t
            nh = _splat_to_scalar(plsc.all_reduce_population_count(hitv))
            ff = _splat_to_scalar(plsc.all_reduce_ffs(hitv))
            first = (found == 0) & (nh > 0)
            excl = csum - g
            cb_here = total + jnp.sum(jnp.where(lanes == ff, excl, 0))
            bstar = jnp.where(first, j * 16 + ff, bstar)
            cumbefore = jnp.where(first, cb_here, cumbefore)
            found = jnp.where(first, np.int32(1), found)
            total = total + jnp.sum(g)
            return (total, found, bstar, cumbefore)

        init = (np.int32(0), np.int32(0), np.int32(0), np.int32(0))
        _, _, bstar, cumbefore = lax.fori_loop(0, 16, sstep, init)
        t = t - cumbefore
        prefix = lax.shift_left(prefix, 8) | bstar
        plsc.subcore_barrier()

    thresh = prefix ^ _MIN32

    for j in range(16):
        wq_v[pl.ds(j * 16, 16)] = maxv16

    gbase = chunk * _CHUNK

    def estep(j, off):
        o = j * 16
        qv = q_v[pl.ds(o, 16)]
        qs = qv ^ _MIN32
        selm = qs <= thresh
        selc = jnp.where(selm, 1, 0)
        csum = plsc.cumsum(selc)
        pos = jnp.minimum(off + (csum - selc), 255)
        plsc.store_scatter(wq_v, [pos], qs, mask=selm)
        plsc.store_scatter(wi_v, [pos], gbase + o + lanes, mask=selm)
        plsc.store_scatter(wx_v, [pos], xs_v[pl.ds(o, 16)], mask=selm)
        plsc.store_scatter(wy_v, [pos], ys_v[pl.ds(o, 16)], mask=selm)
        return off + jnp.max(csum)

    cnt = lax.fori_loop(0, _STEPS, estep, np.int32(0))
    cnt16_v[pl.ds(0, 16)] = jnp.broadcast_to(cnt, (16,)).astype(i32)
    pltpu.sync_copy(wq_v, wq_sh.at[c, s])
    pltpu.sync_copy(wi_v, wi_sh.at[c, s])
    pltpu.sync_copy(wx_v, wx_sh.at[c, s])
    pltpu.sync_copy(wy_v, wy_sh.at[c, s])
    pltpu.sync_copy(cnt16_v, cnt_sh.at[c, s])
    plsc.subcore_barrier()

    @pl.when(chunk == 0)
    def _():
        pltpu.sync_copy(wq_sh.at[c, pl.ds(row0, 8)], m8_v)
        pltpu.sync_copy(wi_sh.at[c, pl.ds(row0, 8)], mi8_v)
        pltpu.sync_copy(wx_sh.at[c, pl.ds(row0, 8)], mx8_v)
        pltpu.sync_copy(wy_sh.at[c, pl.ds(row0, 8)], my8_v)
        pltpu.sync_copy(cnt_sh.at[c, pl.ds(row0, 8)], c8_v)
        cvoff = pl.multiple_of(batch * 8, 8)
        pltpu.sync_copy(cx_hbm.at[pl.ds(cvoff, 8)], covx_v.at[pl.ds(0, 8)])
        pltpu.sync_copy(cy_hbm.at[pl.ds(cvoff, 8)], covy_v.at[pl.ds(0, 8)])
        for j in range(16):
            cq_v[pl.ds(j * 16, 16)] = maxv16
        acc = np.int32(0)
        for tl in range(8):
            cnt_t = jnp.max(c8_v[tl])
            for j in range(16):
                pin = j * 16 + lanes
                msk = pin < cnt_t
                pos = jnp.minimum(acc + pin, 255)
                plsc.store_scatter(cq_v, [pos], m8_v[tl, pl.ds(j * 16, 16)], mask=msk)
                plsc.store_scatter(ci_v, [pos], mi8_v[tl, pl.ds(j * 16, 16)], mask=msk)
                plsc.store_scatter(cx_v, [pos], mx8_v[tl, pl.ds(j * 16, 16)], mask=msk)
                plsc.store_scatter(cy_v, [pos], my8_v[tl, pl.ds(j * 16, 16)], mask=msk)
            acc = acc + cnt_t
        for j in range(8):
            px_v[pl.ds(j * 16, 16)] = jnp.zeros((16,), f32)
            py_v[pl.ds(j * 16, 16)] = jnp.zeros((16,), f32)

        def sortstep(n, carry):
            macc = maxv16
            for j in range(16):
                macc = jnp.minimum(macc, cq_v[pl.ds(j * 16, 16)])
            qmin = jnp.min(macc)
            iacc = maxv16
            for j in range(16):
                v = cq_v[pl.ds(j * 16, 16)]
                iv = ci_v[pl.ds(j * 16, 16)]
                iacc = jnp.minimum(iacc, jnp.where(v == qmin, iv, _MAX32))
            gmin = jnp.min(iacc)
            xacc = jnp.float32(0.0)
            yacc = jnp.float32(0.0)
            for j in range(16):
                v = cq_v[pl.ds(j * 16, 16)]
                iv = ci_v[pl.ds(j * 16, 16)]
                hit = (v == qmin) & (iv == gmin)
                xacc = xacc + jnp.sum(jnp.where(hit, cx_v[pl.ds(j * 16, 16)], 0.0))
                yacc = yacc + jnp.sum(jnp.where(hit, cy_v[pl.ds(j * 16, 16)], 0.0))
                plsc.store_scatter(cq_v, [j * 16 + lanes], maxv16, mask=hit)
            nn = jnp.broadcast_to(n, (16,)).astype(i32)
            plsc.store_scatter(px_v, [nn], jnp.broadcast_to(xacc, (16,)), mask=lanes == 0)
            plsc.store_scatter(py_v, [nn], jnp.broadcast_to(yacc, (16,)), mask=lanes == 0)
            return carry

        lax.fori_loop(0, _NSEL, sortstep, np.int32(0))
        plsc.store_scatter(px_v, [_NSEL + lanes], covx_v[pl.ds(0, 16)],
                           mask=lanes < _NPTS - _NSEL)
        plsc.store_scatter(py_v, [_NSEL + lanes], covy_v[pl.ds(0, 16)],
                           mask=lanes < _NPTS - _NSEL)
        pltpu.sync_copy(px_v, px_sh.at[c, half])
        pltpu.sync_copy(py_v, py_sh.at[c, half])
        out_off = pl.multiple_of(batch * 128, 8)
        pltpu.sync_copy(px_v, px_hbm.at[pl.ds(out_off, 128)])
        pltpu.sync_copy(py_v, py_hbm.at[pl.ds(out_off, 128)])

    plsc.subcore_barrier()

    pltpu.sync_copy(mk_hbm.at[pl.ds(ms_off, 2 * _HW)], ms_v)
    pltpu.sync_copy(px_sh.at[c, half, pl.ds(chunk * 16, 16)], pxw_v)
    pltpu.sync_copy(py_sh.at[c, half, pl.ds(chunk * 16, 16)], pyw_v)
    for j in range(32):
        ctab_v[pl.ds(j * 16, 16)] = (j * 16 + lanes) * _HW

    xo = pxw_v[pl.ds(0, 16)]
    yo = pyw_v[pl.ds(0, 16)]
    lin, fv, wv = tap_vectors(xo, yo)
    ev = tuple((fv[tt] * wv[tt]) for tt in range(4))

    def coarse(off):
        a0 = plsc.load_gather(ms_v, [lin[0] + off]) * ev[0]
        a1 = plsc.load_gather(ms_v, [lin[1] + off]) * ev[1]
        a2 = plsc.load_gather(ms_v, [lin[2] + off]) * ev[2]
        a3 = plsc.load_gather(ms_v, [lin[3] + off]) * ev[3]
        return ((a0 + a1) + a2) + a3

    c0v = coarse(0)
    c1v = coarse(_HW)
    fbase = batch * (_CF * _HW)

    def pstep(p, carry):
        def exi(v):
            return jnp.sum(jnp.where(lanes == p, v, 0))

        def exf(v):
            return jnp.sum(jnp.where(lanes == p, v, jnp.float32(0.0)))

        ls = [exi(lin[tt]) for tt in range(4)]
        es = [exf(ev[tt]) for tt in range(4)]
        for tt in range(4):
            bt = fbase + ls[tt]
            for jj in range(4):
                for q2 in range(8):
                    idxb_v[tt * 4 + jj, pl.ds(q2 * 16, 16)] = (
                        ctab_v[pl.ds(jj * 128 + q2 * 16, 16)] + bt)
        handles = []
        for r in range(16):
            handles.append(pltpu.async_copy(
                ft_hbm.at[idxb_v.at[r]],
                gbuf_v.at[pl.ds(r * 128, 128)], sem))
        for h in handles:
            h.wait()
        for j in range(32):
            g0 = gbuf_v[pl.ds(j * 16, 16)]
            g1 = gbuf_v[pl.ds(_CF + j * 16, 16)]
            g2 = gbuf_v[pl.ds(2 * _CF + j * 16, 16)]
            g3 = gbuf_v[pl.ds(3 * _CF + j * 16, 16)]
            fin = ((g0 * es[0] + g1 * es[1]) + g2 * es[2]) + g3 * es[3]
            rep16_v[pl.ds(p * _ROW + j * 16, 16)] = fin
        tail = (jnp.where(lanes == 0, exf(c0v), 0.0)
                + jnp.where(lanes == 1, exf(c1v), 0.0))
        rep16_v[pl.ds(p * _ROW + _CF, 16)] = tail
        return carry

    lax.fori_loop(0, 16, pstep, np.int32(0))
    rep_off = pl.multiple_of((batch * 128 + chunk * 16) * _ROW, 8)
    pltpu.sync_copy(rep16_v, rep_hbm.at[pl.ds(rep_off, 16 * _ROW)])


_sc_fused = functools.partial(
    pl.kernel,
    out_type=[
        jax.ShapeDtypeStruct((_B * 128,), jnp.float32),
        jax.ShapeDtypeStruct((_B * 128,), jnp.float32),
        jax.ShapeDtypeStruct((_B * 128 * _ROW,), jnp.float32),
    ],
    mesh=plsc.VectorSubcoreMesh(core_axis_name="c", subcore_axis_name="s"),
    compiler_params=pltpu.CompilerParams(needs_layout_passes=False),
    scratch_types=[
        pltpu.VMEM((2 * _HW,), jnp.float32),
        pltpu.VMEM((_CHUNK,), jnp.float32),
        pltpu.VMEM((_CHUNK,), jnp.float32),
        pltpu.VMEM((_CHUNK,), jnp.int32),
        pltpu.VMEM((4096,), jnp.int32),
        pltpu.VMEM((256,), jnp.int32),
        pltpu.VMEM((8, 256), jnp.int32),
        pltpu.VMEM((8, 256), jnp.int32),
        pltpu.VMEM((8, 256), jnp.float32),
        pltpu.VMEM((8, 256), jnp.float32),
        pltpu.VMEM((256,), jnp.int32),
        pltpu.VMEM((256,), jnp.int32),
        pltpu.VMEM((256,), jnp.float32),
        pltpu.VMEM((256,), jnp.float32),
        pltpu.VMEM((256,), jnp.int32),
        pltpu.VMEM((256,), jnp.int32),
        pltpu.VMEM((256,), jnp.float32),
        pltpu.VMEM((256,), jnp.float32),
        pltpu.VMEM((16,), jnp.int32),
        pltpu.VMEM((8, 16), jnp.int32),
        pltpu.VMEM((128,), jnp.float32),
        pltpu.VMEM((128,), jnp.float32),
        pltpu.VMEM((16,), jnp.float32),
        pltpu.VMEM((16,), jnp.float32),
        pltpu.VMEM((16,), jnp.float32),
        pltpu.VMEM((16,), jnp.float32),
        pltpu.VMEM((512,), jnp.int32),
        pltpu.VMEM((16, 128), jnp.int32),
        pltpu.VMEM((2048,), jnp.float32),
        pltpu.VMEM((16 * _ROW,), jnp.float32),
        pltpu.SemaphoreType.DMA,
        pltpu.VMEM_SHARED((2, 16, 256), jnp.int32),
        pltpu.VMEM_SHARED((2, 16, 256), jnp.int32),
        pltpu.VMEM_SHARED((2, 16, 256), jnp.int32),
        pltpu.VMEM_SHARED((2, 16, 256), jnp.float32),
        pltpu.VMEM_SHARED((2, 16, 256), jnp.float32),
        pltpu.VMEM_SHARED((2, 16, 16), jnp.int32),
        pltpu.VMEM_SHARED((2, 2, 128), jnp.float32),
        pltpu.VMEM_SHARED((2, 2, 128), jnp.float32),
    ],
)(_sc_fused_body)


def _mlp_body(rep_ref, w1_ref, w2_ref, w3_ref, w4_ref, b4_ref, out_ref):
    r = rep_ref[0][:, :514]
    h = jnp.maximum(jnp.dot(r, w1_ref[...], preferred_element_type=jnp.float32), 0.0)
    h = jnp.maximum(jnp.dot(h, w2_ref[...], preferred_element_type=jnp.float32), 0.0)
    h = jnp.maximum(jnp.dot(h, w3_ref[...], preferred_element_type=jnp.float32), 0.0)
    out_ref[0] = jnp.dot(h, w4_ref[...], preferred_element_type=jnp.float32) + b4_ref[...]


def _mlp_tc(rep, W1pT, W2T, W3T, W4T, b4):
    B = rep.shape[0]
    return pl.pallas_call(
        _mlp_body,
        grid=(B,),
        in_specs=[
            pl.BlockSpec((1, 128, _ROW), lambda b: (b, 0, 0)),
            pl.BlockSpec(W1pT.shape, lambda b: (0, 0)),
            pl.BlockSpec(W2T.shape, lambda b: (0, 0)),
            pl.BlockSpec(W3T.shape, lambda b: (0, 0)),
            pl.BlockSpec(W4T.shape, lambda b: (0, 0)),
            pl.BlockSpec((1, _NUM_CLASSES), lambda b: (0, 0)),
        ],
        out_specs=pl.BlockSpec((1, 128, _NUM_CLASSES), lambda b: (b, 0, 0)),
        out_shape=jax.ShapeDtypeStruct((B, 128, _NUM_CLASSES), jnp.float32),
    )(rep, W1pT, W2T, W3T, W4T, b4.reshape(1, _NUM_CLASSES))


def kernel(x, feature, mask, W1, W2, W3, W4, b4):
    mask_sm = jax.nn.softmax(mask, axis=1)
    msort = -jnp.sort(-mask_sm, axis=1)
    px, py, rep = _sc_fused(
        msort.reshape(_B * 2 * _HW),
        jnp.asarray(_XS_NP), jnp.asarray(_YS_NP),
        mask.reshape(_B * 2 * _HW),
        feature.reshape(_B * _CF * _HW),
        jnp.asarray(_CVX_NP), jnp.asarray(_CVY_NP),
    )
    pxm = px.reshape(_B, 128)[:, :_NPTS]
    pym = py.reshape(_B, 128)[:, :_NPTS]
    points = jnp.stack([pxm, pym], axis=-1)
    W1pT = jnp.concatenate([W1[:, _NUM_CLASSES:], W1[:, :_NUM_CLASSES]], axis=1).T
    rep3 = rep.reshape(_B, 128, _ROW)
    rend_t = _mlp_tc(rep3, W1pT, W2.T, W3.T, W4.T, b4)
    rend = rend_t[:, :_NPTS, :].transpose(0, 2, 1)
    return (rend, points, mask)


# speedup vs baseline: 1.0115x; 1.0115x over previous
import functools

import jax
import jax.numpy as jnp
import numpy as np
from jax import lax
from jax.experimental import pallas as pl
from jax.experimental.pallas import tpu as pltpu
from jax.experimental.pallas import tpu_sc as plsc

_NUM_CLASSES = 2
_NPTS = 100
_NSEL = 95
_B = 4
_KP = 80000
_G = 128
_HW = _G * _G
_CF = 512
_ROW = 528
_CHUNK = _KP // 8
_STEPS = _CHUNK // 16
_MSK31 = np.int32(0x7FFFFFFF)
_MIN32 = np.int32(-2147483648)
_MAX32 = np.int32(2147483647)


def _fixed_draws():
    k1, k2 = jax.random.split(jax.random.key(42))
    over = jax.random.uniform(k1, (_B, _KP, 2), dtype=jnp.float32)
    cov = jax.random.uniform(k2, (_B, _NPTS - _NSEL, 2), dtype=jnp.float32)
    return (np.asarray(over), np.asarray(cov))


try:
    with jax.default_device(jax.local_devices(backend="cpu")[0]):
        _OVER_NP, _COV_NP = _fixed_draws()
except Exception:
    try:
        _OVER_NP, _COV_NP = _fixed_draws()
    except Exception:
        _OVER_NP = np.zeros((_B, _KP, 2), np.float32)
        _COV_NP = np.zeros((_B, _NPTS - _NSEL, 2), np.float32)

_XS_NP = np.ascontiguousarray(_OVER_NP[..., 0]).reshape(-1)
_YS_NP = np.ascontiguousarray(_OVER_NP[..., 1]).reshape(-1)
_CVX_NP = np.zeros((_B, 8), np.float32)
_CVY_NP = np.zeros((_B, 8), np.float32)
_CVX_NP[:, : _NPTS - _NSEL] = _COV_NP[..., 0]
_CVY_NP[:, : _NPTS - _NSEL] = _COV_NP[..., 1]
_CVX_NP = _CVX_NP.reshape(-1)
_CVY_NP = _CVY_NP.reshape(-1)


def _splat_to_scalar(v):
    return jnp.max(v) if getattr(v, "ndim", 0) else v


def _sc_fused_body(ms_hbm, xs_hbm, ys_hbm, mk_hbm, ft_hbm, cx_hbm, cy_hbm,
                   px_hbm, py_hbm, rep_hbm,
                   ms_v, xs_v, ys_v, q_v, hist_v, hm_v,
                   m8_v, mi8_v, mx8_v, my8_v,
                   wq_v, wi_v, wx_v, wy_v,
                   cq_v, ci_v, cx_v, cy_v,
                   cnt16_v, c8_v, px_v, py_v, covx_v, covy_v,
                   pxw_v, pyw_v, ctab_v, idxb_v, gbuf_v, rep16_v, sem,
                   hist_sh, wq_sh, wi_sh, wx_sh, wy_sh, cnt_sh,
                   px_sh, py_sh):
    i32 = jnp.int32
    f32 = jnp.float32
    c = lax.axis_index("c")
    s = lax.axis_index("s")
    half = s // 8
    chunk = s % 8
    batch = c * 2 + half
    row0 = half * 8
    lanes = lax.iota(i32, 16)
    zeros16 = jnp.zeros((16,), i32)
    maxv16 = zeros16 + _MAX32

    ms_off = pl.multiple_of(batch * (2 * _HW), 8)
    pltpu.sync_copy(ms_hbm.at[pl.ds(ms_off, 2 * _HW)], ms_v)
    off_in = pl.multiple_of(batch * _KP + chunk * _CHUNK, 8)
    pltpu.sync_copy(xs_hbm.at[pl.ds(off_in, _CHUNK)], xs_v)
    pltpu.sync_copy(ys_hbm.at[pl.ds(off_in, _CHUNK)], ys_v)

    def tap_vectors(xo, yo):
        gx1 = (2.0 * xo - 1.0) + 1.0
        gy1 = (2.0 * yo - 1.0) + 1.0
        ix = (gx1 * 128.0 - 1.0) / 2.0
        iy = (gy1 * 128.0 - 1.0) / 2.0
        itx = ix.astype(i32)
        ity = iy.astype(i32)
        ix0 = itx - jnp.where(ix < itx.astype(f32), 1, 0)
        iy0 = ity - jnp.where(iy < ity.astype(f32), 1, 0)
        wx1 = ix - ix0.astype(f32)
        wx0 = 1.0 - wx1
        wy1 = iy - iy0.astype(f32)
        wy0 = 1.0 - wy1
        ix1 = ix0 + 1
        iy1 = iy0 + 1
        vx0 = (ix0 >= 0) & (ix0 <= _G - 1)
        vx1 = (ix1 >= 0) & (ix1 <= _G - 1)
        vy0 = (iy0 >= 0) & (iy0 <= _G - 1)
        vy1 = (iy1 >= 0) & (iy1 <= _G - 1)
        xc0 = jnp.clip(ix0, 0, _G - 1)
        xc1 = jnp.clip(ix1, 0, _G - 1)
        yc0 = jnp.clip(iy0, 0, _G - 1) * _G
        yc1 = jnp.clip(iy1, 0, _G - 1) * _G
        lin = (yc0 + xc0, yc0 + xc1, yc1 + xc0, yc1 + xc1)
        fv = (jnp.where(vx0 & vy0, 1.0, 0.0), jnp.where(vx1 & vy0, 1.0, 0.0),
              jnp.where(vx0 & vy1, 1.0, 0.0), jnp.where(vx1 & vy1, 1.0, 0.0))
        wv = (wx0 * wy0, wx1 * wy0, wx0 * wy1, wx1 * wy1)
        return lin, fv, wv

    def u_step(i, carry):
        o = i * 16
        xo = xs_v[pl.ds(o, 16)]
        yo = ys_v[pl.ds(o, 16)]
        lin, fv, wv = tap_vectors(xo, yo)

        def chan(off):
            a0 = (plsc.load_gather(ms_v, [lin[0] + off]) * fv[0]) * wv[0]
            a1 = (plsc.load_gather(ms_v, [lin[1] + off]) * fv[1]) * wv[1]
            a2 = (plsc.load_gather(ms_v, [lin[2] + off]) * fv[2]) * wv[2]
            a3 = (plsc.load_gather(ms_v, [lin[3] + off]) * fv[3]) * wv[3]
            return ((a0 + a1) + a2) + a3

        u = -1.0 * (chan(0) - chan(_HW))
        bb = lax.bitcast_convert_type(u, i32)
        aa = bb ^ (lax.shift_right_arithmetic(bb, 31) & _MSK31)
        qb = (~aa) ^ _MIN32
        q_v[pl.ds(o, 16)] = qb
        return carry

    lax.fori_loop(0, _STEPS, u_step, np.int32(0))

    t = np.int32(_NSEL)
    prefix = np.int32(0)
    for rnd in range(4):
        shift = 24 - 8 * rnd

        def zstep(j, carry):
            hist_v[pl.ds(j * 16, 16)] = zeros16
            return carry

        lax.fori_loop(0, 256, zstep, np.int32(0))

        def hstep(j, carry):
            qv = q_v[pl.ds(j * 16, 16)]
            binv = lax.shift_right_logical(qv, shift) & 255
            hidx = lanes * 256 + binv
            if rnd == 0:
                cnt = plsc.load_gather(hist_v, [hidx])
                plsc.store_scatter(hist_v, [hidx], cnt + 1)
            else:
                act = lax.shift_right_logical(qv, shift + 8) == carry
                cnt = plsc.load_gather(hist_v, [hidx], mask=act)
                plsc.store_scatter(hist_v, [hidx], cnt + 1, mask=act)
            return carry

        lax.fori_loop(0, _STEPS, hstep, prefix)

        def mstep(j, carry):
            acc = zeros16
            for l in range(16):
                acc = acc + hist_v[pl.ds(l * 256 + j * 16, 16)]
            hm_v[pl.ds(j * 16, 16)] = acc
            return carry

        lax.fori_loop(0, 16, mstep, np.int32(0))
        pltpu.sync_copy(hm_v, hist_sh.at[c, s])
        plsc.subcore_barrier()
        pltpu.sync_copy(hist_sh.at[c, pl.ds(row0, 8)], m8_v)

        def sstep(j, carry):
            total, found, bstar, cumbefore = carry
            g = zeros16
            for l in range(8):
                g = g + m8_v[l, pl.ds(j * 16, 16)]
            csum = plsc.cumsum(g)
            full = total + csum
            hitv = full >= t
            nh = _splat_to_scalar(plsc.all_reduce_population_count(hitv))
            ff = _splat_to_scalar(plsc.all_reduce_ffs(hitv))
            first = (found == 0) & (nh > 0)
            excl = csum - g
            cb_here = total + jnp.sum(jnp.where(lanes == ff, excl, 0))
            bstar = jnp.where(first, j * 16 + ff, bstar)
            cumbefore = jnp.where(first, cb_here, cumbefore)
            found = jnp.where(first, np.int32(1), found)
            total = total + jnp.sum(g)
            return (total, found, bstar, cumbefore)

        init = (np.int32(0), np.int32(0), np.int32(0), np.int32(0))
        _, _, bstar, cumbefore = lax.fori_loop(0, 16, sstep, init)
        t = t - cumbefore
        prefix = lax.shift_left(prefix, 8) | bstar
        plsc.subcore_barrier()

    thresh = prefix ^ _MIN32

    for j in range(16):
        wq_v[pl.ds(j * 16, 16)] = maxv16

    gbase = chunk * _CHUNK

    def estep(j, off):
        o = j * 16
        qv = q_v[pl.ds(o, 16)]
        qs = qv ^ _MIN32
        selm = qs <= thresh
        selc = jnp.where(selm, 1, 0)
        csum = plsc.cumsum(selc)
        pos = jnp.minimum(off + (csum - selc), 255)
        plsc.store_scatter(wq_v, [pos], qs, mask=selm)
        plsc.store_scatter(wi_v, [pos], gbase + o + lanes, mask=selm)
        plsc.store_scatter(wx_v, [pos], xs_v[pl.ds(o, 16)], mask=selm)
        plsc.store_scatter(wy_v, [pos], ys_v[pl.ds(o, 16)], mask=selm)
        return off + jnp.max(csum)

    cnt = lax.fori_loop(0, _STEPS, estep, np.int32(0))
    cnt16_v[pl.ds(0, 16)] = jnp.broadcast_to(cnt, (16,)).astype(i32)
    pltpu.sync_copy(wq_v, wq_sh.at[c, s])
    pltpu.sync_copy(wi_v, wi_sh.at[c, s])
    pltpu.sync_copy(wx_v, wx_sh.at[c, s])
    pltpu.sync_copy(wy_v, wy_sh.at[c, s])
    pltpu.sync_copy(cnt16_v, cnt_sh.at[c, s])
    plsc.subcore_barrier()

    @pl.when(chunk == 0)
    def _():
        pltpu.sync_copy(wq_sh.at[c, pl.ds(row0, 8)], m8_v)
        pltpu.sync_copy(wi_sh.at[c, pl.ds(row0, 8)], mi8_v)
        pltpu.sync_copy(wx_sh.at[c, pl.ds(row0, 8)], mx8_v)
        pltpu.sync_copy(wy_sh.at[c, pl.ds(row0, 8)], my8_v)
        pltpu.sync_copy(cnt_sh.at[c, pl.ds(row0, 8)], c8_v)
        cvoff = pl.multiple_of(batch * 8, 8)
        pltpu.sync_copy(cx_hbm.at[pl.ds(cvoff, 8)], covx_v.at[pl.ds(0, 8)])
        pltpu.sync_copy(cy_hbm.at[pl.ds(cvoff, 8)], covy_v.at[pl.ds(0, 8)])
        for j in range(16):
            cq_v[pl.ds(j * 16, 16)] = maxv16
        acc = np.int32(0)
        for tl in range(8):
            cnt_t = jnp.max(c8_v[tl])
            for j in range(16):
                pin = j * 16 + lanes
                msk = pin < cnt_t
                pos = jnp.minimum(acc + pin, 255)
                plsc.store_scatter(cq_v, [pos], m8_v[tl, pl.ds(j * 16, 16)], mask=msk)
                plsc.store_scatter(ci_v, [pos], mi8_v[tl, pl.ds(j * 16, 16)], mask=msk)
                plsc.store_scatter(cx_v, [pos], mx8_v[tl, pl.ds(j * 16, 16)], mask=msk)
                plsc.store_scatter(cy_v, [pos], my8_v[tl, pl.ds(j * 16, 16)], mask=msk)
            acc = acc + cnt_t
        for j in range(8):
            px_v[pl.ds(j * 16, 16)] = jnp.zeros((16,), f32)
            py_v[pl.ds(j * 16, 16)] = jnp.zeros((16,), f32)

        def sortstep(n, carry):
            macc = maxv16
            for j in range(16):
                macc = jnp.minimum(macc, cq_v[pl.ds(j * 16, 16)])
            qmin = jnp.min(macc)
            iacc = maxv16
            for j in range(16):
                v = cq_v[pl.ds(j * 16, 16)]
                iv = ci_v[pl.ds(j * 16, 16)]
                iacc = jnp.minimum(iacc, jnp.where(v == qmin, iv, _MAX32))
            gmin = jnp.min(iacc)
            xacc = jnp.float32(0.0)
            yacc = jnp.float32(0.0)
            for j in range(16):
                v = cq_v[pl.ds(j * 16, 16)]
                iv = ci_v[pl.ds(j * 16, 16)]
                hit = (v == qmin) & (iv == gmin)
                xacc = xacc + jnp.sum(jnp.where(hit, cx_v[pl.ds(j * 16, 16)], 0.0))
                yacc = yacc + jnp.sum(jnp.where(hit, cy_v[pl.ds(j * 16, 16)], 0.0))
                plsc.store_scatter(cq_v, [j * 16 + lanes], maxv16, mask=hit)
            nn = jnp.broadcast_to(n, (16,)).astype(i32)
            plsc.store_scatter(px_v, [nn], jnp.broadcast_to(xacc, (16,)), mask=lanes == 0)
            plsc.store_scatter(py_v, [nn], jnp.broadcast_to(yacc, (16,)), mask=lanes == 0)
            return carry

        lax.fori_loop(0, _NSEL, sortstep, np.int32(0))
        plsc.store_scatter(px_v, [_NSEL + lanes], covx_v[pl.ds(0, 16)],
                           mask=lanes < _NPTS - _NSEL)
        plsc.store_scatter(py_v, [_NSEL + lanes], covy_v[pl.ds(0, 16)],
                           mask=lanes < _NPTS - _NSEL)
        pltpu.sync_copy(px_v, px_sh.at[c, half])
        pltpu.sync_copy(py_v, py_sh.at[c, half])
        out_off = pl.multiple_of(batch * 128, 8)
        pltpu.sync_copy(px_v, px_hbm.at[pl.ds(out_off, 128)])
        pltpu.sync_copy(py_v, py_hbm.at[pl.ds(out_off, 128)])

    plsc.subcore_barrier()

    pltpu.sync_copy(mk_hbm.at[pl.ds(ms_off, 2 * _HW)], ms_v)
    pltpu.sync_copy(px_sh.at[c, half, pl.ds(chunk * 16, 16)], pxw_v)
    pltpu.sync_copy(py_sh.at[c, half, pl.ds(chunk * 16, 16)], pyw_v)
    for j in range(32):
        ctab_v[pl.ds(j * 16, 16)] = (j * 16 + lanes) * _HW

    xo = pxw_v[pl.ds(0, 16)]
    yo = pyw_v[pl.ds(0, 16)]
    lin, fv, wv = tap_vectors(xo, yo)
    ev = tuple((fv[tt] * wv[tt]) for tt in range(4))

    def coarse(off):
        a0 = plsc.load_gather(ms_v, [lin[0] + off]) * ev[0]
        a1 = plsc.load_gather(ms_v, [lin[1] + off]) * ev[1]
        a2 = plsc.load_gather(ms_v, [lin[2] + off]) * ev[2]
        a3 = plsc.load_gather(ms_v, [lin[3] + off]) * ev[3]
        return ((a0 + a1) + a2) + a3

    c0v = coarse(0)
    c1v = coarse(_HW)
    fbase = batch * (_CF * _HW)

    def pstep(p, carry):
        pb = lax.rem(p, 2)

        @pl.when(p < 16)
        def _issue():
            def exi(v):
                return jnp.sum(jnp.where(lanes == p, v, 0))

            ls = [exi(lin[tt]) for tt in range(4)]
            for tt in range(4):
                bt = fbase + ls[tt]
                for jj in range(4):
                    for q2 in range(8):
                        idxb_v[pb * 16 + tt * 4 + jj, pl.ds(q2 * 16, 16)] = (
                            ctab_v[pl.ds(jj * 128 + q2 * 16, 16)] + bt)
            for r in range(16):
                pltpu.async_copy(
                    ft_hbm.at[idxb_v.at[pb * 16 + r]],
                    gbuf_v.at[pl.ds(pb * 2048 + r * 128, 128)], sem)

        @pl.when(p > 0)
        def _combine():
            pc = p - 1
            qb = lax.rem(pc, 2)

            def exf(v):
                return jnp.sum(jnp.where(lanes == pc, v, jnp.float32(0.0)))

            for r in range(16):
                pltpu.make_async_copy(
                    ft_hbm.at[idxb_v.at[qb * 16 + r]],
                    gbuf_v.at[pl.ds(qb * 2048 + r * 128, 128)], sem).wait()
            es = [exf(ev[tt]) for tt in range(4)]
            for j in range(32):
                g0 = gbuf_v[pl.ds(qb * 2048 + j * 16, 16)]
                g1 = gbuf_v[pl.ds(qb * 2048 + _CF + j * 16, 16)]
                g2 = gbuf_v[pl.ds(qb * 2048 + 2 * _CF + j * 16, 16)]
                g3 = gbuf_v[pl.ds(qb * 2048 + 3 * _CF + j * 16, 16)]
                fin = ((g0 * es[0] + g1 * es[1]) + g2 * es[2]) + g3 * es[3]
                rep16_v[pl.ds(pc * _ROW + j * 16, 16)] = fin
            tail = (jnp.where(lanes == 0, exf(c0v), 0.0)
                    + jnp.where(lanes == 1, exf(c1v), 0.0))
            rep16_v[pl.ds(pc * _ROW + _CF, 16)] = tail

        return carry

    lax.fori_loop(0, 17, pstep, np.int32(0))
    rep_off = pl.multiple_of((batch * 128 + chunk * 16) * _ROW, 8)
    pltpu.sync_copy(rep16_v, rep_hbm.at[pl.ds(rep_off, 16 * _ROW)])


_sc_fused = functools.partial(
    pl.kernel,
    out_type=[
        jax.ShapeDtypeStruct((_B * 128,), jnp.float32),
        jax.ShapeDtypeStruct((_B * 128,), jnp.float32),
        jax.ShapeDtypeStruct((_B * 128 * _ROW,), jnp.float32),
    ],
    mesh=plsc.VectorSubcoreMesh(core_axis_name="c", subcore_axis_name="s"),
    compiler_params=pltpu.CompilerParams(needs_layout_passes=False),
    scratch_types=[
        pltpu.VMEM((2 * _HW,), jnp.float32),
        pltpu.VMEM((_CHUNK,), jnp.float32),
        pltpu.VMEM((_CHUNK,), jnp.float32),
        pltpu.VMEM((_CHUNK,), jnp.int32),
        pltpu.VMEM((4096,), jnp.int32),
        pltpu.VMEM((256,), jnp.int32),
        pltpu.VMEM((8, 256), jnp.int32),
        pltpu.VMEM((8, 256), jnp.int32),
        pltpu.VMEM((8, 256), jnp.float32),
        pltpu.VMEM((8, 256), jnp.float32),
        pltpu.VMEM((256,), jnp.int32),
        pltpu.VMEM((256,), jnp.int32),
        pltpu.VMEM((256,), jnp.float32),
        pltpu.VMEM((256,), jnp.float32),
        pltpu.VMEM((256,), jnp.int32),
        pltpu.VMEM((256,), jnp.int32),
        pltpu.VMEM((256,), jnp.float32),
        pltpu.VMEM((256,), jnp.float32),
        pltpu.VMEM((16,), jnp.int32),
        pltpu.VMEM((8, 16), jnp.int32),
        pltpu.VMEM((128,), jnp.float32),
        pltpu.VMEM((128,), jnp.float32),
        pltpu.VMEM((16,), jnp.float32),
        pltpu.VMEM((16,), jnp.float32),
        pltpu.VMEM((16,), jnp.float32),
        pltpu.VMEM((16,), jnp.float32),
        pltpu.VMEM((512,), jnp.int32),
        pltpu.VMEM((32, 128), jnp.int32),
        pltpu.VMEM((4096,), jnp.float32),
        pltpu.VMEM((16 * _ROW,), jnp.float32),
        pltpu.SemaphoreType.DMA,
        pltpu.VMEM_SHARED((2, 16, 256), jnp.int32),
        pltpu.VMEM_SHARED((2, 16, 256), jnp.int32),
        pltpu.VMEM_SHARED((2, 16, 256), jnp.int32),
        pltpu.VMEM_SHARED((2, 16, 256), jnp.float32),
        pltpu.VMEM_SHARED((2, 16, 256), jnp.float32),
        pltpu.VMEM_SHARED((2, 16, 16), jnp.int32),
        pltpu.VMEM_SHARED((2, 2, 128), jnp.float32),
        pltpu.VMEM_SHARED((2, 2, 128), jnp.float32),
    ],
)(_sc_fused_body)


def _mlp_body(rep_ref, w1_ref, w2_ref, w3_ref, w4_ref, b4_ref, out_ref):
    r = rep_ref[0][:, :514]
    h = jnp.maximum(jnp.dot(r, w1_ref[...], preferred_element_type=jnp.float32), 0.0)
    h = jnp.maximum(jnp.dot(h, w2_ref[...], preferred_element_type=jnp.float32), 0.0)
    h = jnp.maximum(jnp.dot(h, w3_ref[...], preferred_element_type=jnp.float32), 0.0)
    out_ref[0] = jnp.dot(h, w4_ref[...], preferred_element_type=jnp.float32) + b4_ref[...]


def _mlp_tc(rep, W1pT, W2T, W3T, W4T, b4):
    B = rep.shape[0]
    return pl.pallas_call(
        _mlp_body,
        grid=(B,),
        in_specs=[
            pl.BlockSpec((1, 128, _ROW), lambda b: (b, 0, 0)),
            pl.BlockSpec(W1pT.shape, lambda b: (0, 0)),
            pl.BlockSpec(W2T.shape, lambda b: (0, 0)),
            pl.BlockSpec(W3T.shape, lambda b: (0, 0)),
            pl.BlockSpec(W4T.shape, lambda b: (0, 0)),
            pl.BlockSpec((1, _NUM_CLASSES), lambda b: (0, 0)),
        ],
        out_specs=pl.BlockSpec((1, 128, _NUM_CLASSES), lambda b: (b, 0, 0)),
        out_shape=jax.ShapeDtypeStruct((B, 128, _NUM_CLASSES), jnp.float32),
    )(rep, W1pT, W2T, W3T, W4T, b4.reshape(1, _NUM_CLASSES))


def kernel(x, feature, mask, W1, W2, W3, W4, b4):
    mask_sm = jax.nn.softmax(mask, axis=1)
    msort = -jnp.sort(-mask_sm, axis=1)
    px, py, rep = _sc_fused(
        msort.reshape(_B * 2 * _HW),
        jnp.asarray(_XS_NP), jnp.asarray(_YS_NP),
        mask.reshape(_B * 2 * _HW),
        feature.reshape(_B * _CF * _HW),
        jnp.asarray(_CVX_NP), jnp.asarray(_CVY_NP),
    )
    pxm = px.reshape(_B, 128)[:, :_NPTS]
    pym = py.reshape(_B, 128)[:, :_NPTS]
    points = jnp.stack([pxm, pym], axis=-1)
    W1pT = jnp.concatenate([W1[:, _NUM_CLASSES:], W1[:, :_NUM_CLASSES]], axis=1).T
    rep3 = rep.reshape(_B, 128, _ROW)
    rend_t = _mlp_tc(rep3, W1pT, W2.T, W3.T, W4.T, b4)
    rend = rend_t[:, :_NPTS, :].transpose(0, 2, 1)
    return (rend, points, mask)
